# Initial kernel scaffold; baseline (speedup 1.0000x reference)
import functools

import jax
import jax.numpy as jnp
from jax import lax
from jax.experimental import pallas as pl
from jax.experimental.pallas import tpu as pltpu
from jax.experimental.pallas import tpu_sc as plsc

N_UE = 50000
N_AP = 5000
E = 1600000
D = 32
ED = 8

NC = 2
NS = 16
NW = NC * NS
EW = E // NW
CH = 80
NCH = EW // CH

f32 = jnp.float32


def _relu(x):
    return jnp.maximum(x, 0.0)


def _mesh():
    return plsc.VectorSubcoreMesh(
        core_axis_name="c", subcore_axis_name="s", num_cores=NC, num_subcores=NS
    )


@functools.cache
def _gather_kernel(d):
    def body(table, idx2, out, idx_v, rows_v, sem):
        c = lax.axis_index("c")
        s = lax.axis_index("s")
        wid = c * NS + s
        pltpu.sync_copy(idx2.at[pl.ds(wid * NCH, NCH)], idx_v)

        def step(ci, carry):
            pltpu.async_copy(table.at[idx_v.at[ci]], rows_v, sem).wait()
            pltpu.sync_copy(rows_v, out.at[pl.ds(wid * EW + ci * CH, CH)])
            return carry

        lax.fori_loop(0, NCH, step, 0, unroll=False)

    return pl.kernel(
        body,
        out_type=jax.ShapeDtypeStruct((E, d), f32),
        mesh=_mesh(),
        scratch_types=[
            pltpu.VMEM((NCH, CH), jnp.int32),
            pltpu.VMEM((CH, d), f32),
            pltpu.SemaphoreType.DMA,
        ],
    )


def _sc_gather(table, idx2):
    return _gather_kernel(int(table.shape[1]))(table, idx2)


@functools.cache
def _scatter_kernel(n):
    def body(vals, idx2, zeros, out, idx_v, rows_v, acc):
        c = lax.axis_index("c")
        s = lax.axis_index("s")
        wid = c * NS + s

        @pl.when(s == 0)
        def _zero():
            pltpu.sync_copy(zeros, acc)

        plsc.subcore_barrier()
        pltpu.sync_copy(idx2.at[pl.ds(wid * NCH, NCH)], idx_v)

        def step(ci, carry):
            pltpu.sync_copy(vals.at[pl.ds(wid * EW + ci * CH, CH)], rows_v)
            pltpu.sync_copy(rows_v, acc.at[idx_v.at[ci]], add=True)
            return carry

        lax.fori_loop(0, NCH, step, 0, unroll=False)
        plsc.subcore_barrier()

        @pl.when(s == 0)
        def _writeout():
            pltpu.sync_copy(acc, out.at[c])

    return pl.kernel(
        body,
        out_type=jax.ShapeDtypeStruct((NC, n, D), f32),
        mesh=_mesh(),
        scratch_types=[
            pltpu.VMEM((NCH, CH), jnp.int32),
            pltpu.VMEM((CH, D), f32),
            pltpu.VMEM_SHARED((n, D), f32),
        ],
    )


def _sc_scatter(vals, idx2, n):
    zeros = jnp.zeros((n, D), f32)
    return _scatter_kernel(n)(vals, idx2, zeros)


BE = 3200
GE = E // BE
BN_UE = 2000
G_UE = N_UE // BN_UE


def _full(shape):
    return pl.BlockSpec(shape, lambda i: tuple(0 for _ in shape))


def _rows(shape):
    return pl.BlockSpec(shape, lambda i: (i,) + tuple(0 for _ in shape[1:]))


def _dot(a, b):
    return jax.lax.dot_general(a, b, (((1,), (0,)), ((), ())), preferred_element_type=f32)


def _tc_node1(x_ue, We1r, Wm1r, Wu1, bu1):
    def body(x, we1r, wm1r, wu1, bu1_, o8, o32, oue):
        xv = x[...]
        o8[...] = xv * we1r[...]
        o32[...] = xv * wm1r[...]
        oue[...] = _relu(xv * wu1[...] + bu1_[...])

    return pl.pallas_call(
        body,
        grid=(G_UE,),
        in_specs=[
            _rows((BN_UE, 1)),
            _full((1, ED)), _full((1, D)), _full((1, D)), _full((1, D)),
        ],
        out_specs=[_rows((BN_UE, ED)), _rows((BN_UE, D)), _rows((BN_UE, D))],
        out_shape=[
            jax.ShapeDtypeStruct((N_UE, ED), f32),
            jax.ShapeDtypeStruct((N_UE, D), f32),
            jax.ShapeDtypeStruct((N_UE, D), f32),
        ],
    )(x_ue, We1r, Wm1r, Wu1, bu1)


def _tc_edge1(g8, g32, ea_u2a, ea_a2u, We1b, be1, Wa1, ba1, Wm1b, bm1):
    def body(g8_, g32_, eu, ea, we1b, be1_, wa1, ba1_, wm1b, bm1_, oe1, oe2, om):
        euv = eu[...]
        e1 = _relu(g8_[...] + euv[:, 0:1] * we1b[0:1, :] + euv[:, 1:2] * we1b[1:2, :] + be1_[...])
        eav = ea[...]
        e2 = _relu(eav[:, 0:1] * wa1[0:1, :] + eav[:, 1:2] * wa1[1:2, :] + ba1_[...])
        om[...] = _relu(g32_[...] + _dot(e1, wm1b[...]) + bm1_[...])
        oe1[...] = e1
        oe2[...] = e2

    return pl.pallas_call(
        body,
        grid=(GE,),
        in_specs=[
            _rows((BE, ED)), _rows((BE, D)), _rows((BE, 2)), _rows((BE, 2)),
            _full((2, ED)), _full((1, ED)), _full((2, ED)), _full((1, ED)),
            _full((ED, D)), _full((1, D)),
        ],
        out_specs=[_rows((BE, ED)), _rows((BE, ED)), _rows((BE, D))],
        out_shape=[
            jax.ShapeDtypeStruct((E, ED), f32),
            jax.ShapeDtypeStruct((E, ED), f32),
            jax.ShapeDtypeStruct((E, D), f32),
        ],
    )(g8, g32, ea_u2a, ea_a2u, We1b, be1, Wa1, ba1, Wm1b, bm1)


def _tc_edge_het(gsu, gda, gsa, gdu, eu, ea, WeA, WeB, WeC, be, WmA, WmB, bm):
    def body(gsu_, gda_, gsa_, gdu_, eu_, ea_, weA, weB, weC, be_, wmA, wmB, bm_,
             oe1, oe2, omap, omue):
        e1 = _relu(_dot(gsu_[...], weA[...]) + _dot(gda_[...], weB[...])
                   + _dot(eu_[...], weC[...]) + be_[...])
        e2 = _relu(_dot(gsa_[...], weA[...]) + _dot(gdu_[...], weB[...])
                   + _dot(ea_[...], weC[...]) + be_[...])
        omap[...] = _relu(_dot(gsu_[...], wmA[...]) + _dot(e1, wmB[...]) + bm_[...])
        omue[...] = _relu(_dot(gsa_[...], wmA[...]) + _dot(e2, wmB[...]) + bm_[...])
        oe1[...] = e1
        oe2[...] = e2

    return pl.pallas_call(
        body,
        grid=(GE,),
        in_specs=[
            _rows((BE, D)), _rows((BE, D)), _rows((BE, D)), _rows((BE, D)),
            _rows((BE, ED)), _rows((BE, ED)),
            _full((D, ED)), _full((D, ED)), _full((ED, ED)), _full((1, ED)),
            _full((D, D)), _full((ED, D)), _full((1, D)),
        ],
        out_specs=[_rows((BE, ED)), _rows((BE, ED)), _rows((BE, D)), _rows((BE, D))],
        out_shape=[
            jax.ShapeDtypeStruct((E, ED), f32),
            jax.ShapeDtypeStruct((E, ED), f32),
            jax.ShapeDtypeStruct((E, D), f32),
            jax.ShapeDtypeStruct((E, D), f32),
        ],
    )(gsu, gda, gsa, gdu, eu, ea, WeA, WeB, WeC, be, WmA, WmB, bm)


def _tc_combine_ap1(parts):
    def body(p, o):
        pv = p[...]
        o[...] = pv[0] + pv[1]

    return pl.pallas_call(
        body,
        grid=(1,),
        in_specs=[_full((NC, N_AP, D))],
        out_specs=_full((N_AP, D)),
        out_shape=jax.ShapeDtypeStruct((N_AP, D), f32),
    )(parts)


def _tc_node_update(x, parts, Ws, bs, n, bn):
    g = n // bn

    def body(x_, p, ws, bs_, o):
        pv = p[...]
        o[...] = _relu(_dot(x_[...], ws[...]) + bs_[...] + pv[0] + pv[1])

    return pl.pallas_call(
        body,
        grid=(g,),
        in_specs=[
            _rows((bn, D)),
            pl.BlockSpec((NC, bn, D), lambda i: (0, i, 0)),
            _full((D, D)), _full((1, D)),
        ],
        out_specs=_rows((bn, D)),
        out_shape=jax.ShapeDtypeStruct((n, D), f32),
    )(x, parts, Ws, bs)


def _tc_node_update_head(x, parts, Ws, bs, Wp1, bp1, Wp2, bp2):
    def body(x_, p, ws, bs_, wp1, bp1_, wp2, bp2_, o, opow):
        pv = p[...]
        ue3 = _relu(_dot(x_[...], ws[...]) + bs_[...] + pv[0] + pv[1])
        h = _relu(_dot(ue3, wp1[...]) + bp1_[...])
        opow[...] = jax.nn.sigmoid(_dot(h, wp2[...]) + bp2_[...])
        o[...] = ue3

    return pl.pallas_call(
        body,
        grid=(G_UE,),
        in_specs=[
            _rows((BN_UE, D)),
            pl.BlockSpec((NC, BN_UE, D), lambda i: (0, i, 0)),
            _full((D, D)), _full((1, D)),
            _full((D, 16)), _full((1, 16)), _full((16, 1)), _full((1, 1)),
        ],
        out_specs=[_rows((BN_UE, D)), _rows((BN_UE, 1))],
        out_shape=[
            jax.ShapeDtypeStruct((N_UE, D), f32),
            jax.ShapeDtypeStruct((N_UE, 1), f32),
        ],
    )(x, parts, Ws, bs, Wp1, bp1, Wp2, bp2)


def kernel(x_ue, x_ap, edge_attr_u2a, edge_attr_a2u, src_ue, dst_ap, src_ap, dst_ue,
           We1, be1, Wa1, ba1, Wm1, bm1, Wu1, bu1,
           We2, be2, Wm2, bm2, Ws2, bs2,
           We3, be3, Wm3, bm3, Ws3, bs3,
           Wp1, bp1, Wp2, bp2):
    i_su = src_ue.astype(jnp.int32).reshape(NW * NCH, CH)
    i_da = dst_ap.astype(jnp.int32).reshape(NW * NCH, CH)
    i_sa = src_ap.astype(jnp.int32).reshape(NW * NCH, CH)
    i_du = dst_ue.astype(jnp.int32).reshape(NW * NCH, CH)

    r = lambda b: b.reshape(1, -1)

    xue8, xue32, ue1 = _tc_node1(x_ue, We1[0:1], Wm1[0:1], Wu1[0:1], r(bu1))
    g8 = _sc_gather(xue8, i_su)
    g32 = _sc_gather(xue32, i_su)
    e_u2a, e_a2u, msg1 = _tc_edge1(
        g8, g32, edge_attr_u2a, edge_attr_a2u,
        We1[1:3], r(be1), Wa1, r(ba1), Wm1[1:9], r(bm1))
    ap_parts = _sc_scatter(msg1, i_da, N_AP)
    ap1 = _tc_combine_ap1(ap_parts)

    def het(ue, ap, eu, ea, We, be, Wm, bm):
        gsu = _sc_gather(ue, i_su)
        gda = _sc_gather(ap, i_da)
        gsa = _sc_gather(ap, i_sa)
        gdu = _sc_gather(ue, i_du)
        e1, e2, m_ap, m_ue = _tc_edge_het(
            gsu, gda, gsa, gdu, eu, ea,
            We[0:32], We[32:64], We[64:72], r(be), Wm[0:32], Wm[32:40], r(bm))
        app = _sc_scatter(m_ap, i_da, N_AP)
        uep = _sc_scatter(m_ue, i_du, N_UE)
        return e1, e2, app, uep

    e_u2a, e_a2u, app, uep = het(ue1, ap1, e_u2a, e_a2u, We2, be2, Wm2, bm2)
    ap2 = _tc_node_update(ap1, app, Ws2, r(bs2), N_AP, 1000)
    ue2 = _tc_node_update(ue1, uep, Ws2, r(bs2), N_UE, BN_UE)

    e_u2a, e_a2u, app, uep = het(ue2, ap2, e_u2a, e_a2u, We3, be3, Wm3, bm3)
    ap3 = _tc_node_update(ap2, app, Ws3, r(bs3), N_AP, 1000)
    ue3, power = _tc_node_update_head(ue2, uep, Ws3, r(bs3), Wp1, r(bp1), Wp2, r(bp2))

    ue_out = jnp.concatenate([ue3[:, :1], power], axis=1)
    return (ue_out, ap3, e_u2a, e_a2u)


# speedup vs baseline: 2.2529x; 2.2529x over previous
import functools

import jax
import jax.numpy as jnp
from jax import lax
from jax.experimental import pallas as pl
from jax.experimental.pallas import tpu as pltpu
from jax.experimental.pallas import tpu_sc as plsc

N_UE = 50000
N_AP = 5000
E = 1600000
D = 32
ED = 8

NC = 2
NS = 16
NW = NC * NS
EW = E // NW
CH = 80
NCH = EW // CH

f32 = jnp.float32


def _relu(x):
    return jnp.maximum(x, 0.0)


def _mesh():
    return plsc.VectorSubcoreMesh(
        core_axis_name="c", subcore_axis_name="s", num_cores=NC, num_subcores=NS
    )


@functools.cache
def _gather_kernel(d):
    def body(table, idx2, out, idx_v, rows_v, sem):
        c = lax.axis_index("c")
        s = lax.axis_index("s")
        wid = c * NS + s
        pltpu.sync_copy(idx2.at[wid], idx_v)

        def step(ci, carry):
            pltpu.async_copy(table.at[idx_v.at[ci]], rows_v, sem).wait()
            pltpu.sync_copy(rows_v, out.at[pl.ds(wid * EW + ci * CH, CH)])
            return carry

        lax.fori_loop(0, NCH, step, 0, unroll=False)

    return pl.kernel(
        body,
        out_type=jax.ShapeDtypeStruct((E, d), f32),
        mesh=_mesh(),
        compiler_params=pltpu.CompilerParams(use_tc_tiling_on_sc=False),
        scratch_types=[
            pltpu.VMEM((NCH, CH), jnp.int32),
            pltpu.VMEM((CH, d), f32),
            pltpu.SemaphoreType.DMA,
        ],
    )


def _sc_gather(table, idx2):
    return _gather_kernel(int(table.shape[1]))(table, idx2)


@functools.cache
def _scatter_kernel(n):
    def body(vals, idx2, zeros, out, idx_v, rows_v, acc):
        c = lax.axis_index("c")
        s = lax.axis_index("s")
        wid = c * NS + s

        @pl.when(s == 0)
        def _zero():
            pltpu.sync_copy(zeros, acc)

        plsc.subcore_barrier()

        def step(ci, carry):
            pltpu.sync_copy(idx2.at[wid, ci], idx_v)
            pltpu.sync_copy(vals.at[pl.ds(wid * EW + ci * CH, CH)], rows_v)
            pltpu.sync_copy(rows_v, acc.at[idx_v], add=True)
            return carry

        lax.fori_loop(0, NCH, step, 0, unroll=False)
        plsc.subcore_barrier()

        @pl.when(s == 0)
        def _writeout():
            pltpu.sync_copy(acc, out.at[c])

    return pl.kernel(
        body,
        out_type=jax.ShapeDtypeStruct((NC, n, D), f32),
        mesh=_mesh(),
        compiler_params=pltpu.CompilerParams(use_tc_tiling_on_sc=False),
        scratch_types=[
            pltpu.VMEM((CH,), jnp.int32),
            pltpu.VMEM((CH, D), f32),
            pltpu.VMEM_SHARED((n, D), f32),
        ],
    )


def _sc_scatter(vals, idx2, n):
    zeros = jnp.zeros((n, D), f32)
    return _scatter_kernel(n)(vals, idx2, zeros)


BE = 3200
GE = E // BE
BN_UE = 2000
G_UE = N_UE // BN_UE


def _full(shape):
    return pl.BlockSpec(shape, lambda i: tuple(0 for _ in shape))


def _rows(shape):
    return pl.BlockSpec(shape, lambda i: (i,) + tuple(0 for _ in shape[1:]))


def _dot(a, b):
    return jax.lax.dot_general(a, b, (((1,), (0,)), ((), ())), preferred_element_type=f32)


def _tc_node1(x_ue, We1r, Wm1r, Wu1, bu1):
    def body(x, we1r, wm1r, wu1, bu1_, o8, o32, oue):
        xv = x[...]
        o8[...] = xv * we1r[...]
        o32[...] = xv * wm1r[...]
        oue[...] = _relu(xv * wu1[...] + bu1_[...])

    return pl.pallas_call(
        body,
        grid=(G_UE,),
        in_specs=[
            _rows((BN_UE, 1)),
            _full((1, ED)), _full((1, D)), _full((1, D)), _full((1, D)),
        ],
        out_specs=[_rows((BN_UE, ED)), _rows((BN_UE, D)), _rows((BN_UE, D))],
        out_shape=[
            jax.ShapeDtypeStruct((N_UE, ED), f32),
            jax.ShapeDtypeStruct((N_UE, D), f32),
            jax.ShapeDtypeStruct((N_UE, D), f32),
        ],
    )(x_ue, We1r, Wm1r, Wu1, bu1)


def _tc_edge1(g8, g32, ea_u2a, ea_a2u, We1b, be1, Wa1, ba1, Wm1b, bm1):
    def body(g8_, g32_, eu, ea, we1b, be1_, wa1, ba1_, wm1b, bm1_, oe1, oe2, om):
        euv = eu[...]
        e1 = _relu(g8_[...] + euv[:, 0:1] * we1b[0:1, :] + euv[:, 1:2] * we1b[1:2, :] + be1_[...])
        eav = ea[...]
        e2 = _relu(eav[:, 0:1] * wa1[0:1, :] + eav[:, 1:2] * wa1[1:2, :] + ba1_[...])
        om[...] = _relu(g32_[...] + _dot(e1, wm1b[...]) + bm1_[...])
        oe1[...] = e1
        oe2[...] = e2

    return pl.pallas_call(
        body,
        grid=(GE,),
        in_specs=[
            _rows((BE, ED)), _rows((BE, D)), _rows((BE, 2)), _rows((BE, 2)),
            _full((2, ED)), _full((1, ED)), _full((2, ED)), _full((1, ED)),
            _full((ED, D)), _full((1, D)),
        ],
        out_specs=[_rows((BE, ED)), _rows((BE, ED)), _rows((BE, D))],
        out_shape=[
            jax.ShapeDtypeStruct((E, ED), f32),
            jax.ShapeDtypeStruct((E, ED), f32),
            jax.ShapeDtypeStruct((E, D), f32),
        ],
    )(g8, g32, ea_u2a, ea_a2u, We1b, be1, Wa1, ba1, Wm1b, bm1)


def _tc_edge_het(gsu, gda, gsa, gdu, eu, ea, WeA, WeB, WeC, be, WmA, WmB, bm):
    def body(gsu_, gda_, gsa_, gdu_, eu_, ea_, weA, weB, weC, be_, wmA, wmB, bm_,
             oe1, oe2, omap, omue):
        e1 = _relu(_dot(gsu_[...], weA[...]) + _dot(gda_[...], weB[...])
                   + _dot(eu_[...], weC[...]) + be_[...])
        e2 = _relu(_dot(gsa_[...], weA[...]) + _dot(gdu_[...], weB[...])
                   + _dot(ea_[...], weC[...]) + be_[...])
        omap[...] = _relu(_dot(gsu_[...], wmA[...]) + _dot(e1, wmB[...]) + bm_[...])
        omue[...] = _relu(_dot(gsa_[...], wmA[...]) + _dot(e2, wmB[...]) + bm_[...])
        oe1[...] = e1
        oe2[...] = e2

    return pl.pallas_call(
        body,
        grid=(GE,),
        in_specs=[
            _rows((BE, D)), _rows((BE, D)), _rows((BE, D)), _rows((BE, D)),
            _rows((BE, ED)), _rows((BE, ED)),
            _full((D, ED)), _full((D, ED)), _full((ED, ED)), _full((1, ED)),
            _full((D, D)), _full((ED, D)), _full((1, D)),
        ],
        out_specs=[_rows((BE, ED)), _rows((BE, ED)), _rows((BE, D)), _rows((BE, D))],
        out_shape=[
            jax.ShapeDtypeStruct((E, ED), f32),
            jax.ShapeDtypeStruct((E, ED), f32),
            jax.ShapeDtypeStruct((E, D), f32),
            jax.ShapeDtypeStruct((E, D), f32),
        ],
    )(gsu, gda, gsa, gdu, eu, ea, WeA, WeB, WeC, be, WmA, WmB, bm)


def _tc_combine_ap1(parts):
    def body(p, o):
        pv = p[...]
        o[...] = pv[0] + pv[1]

    return pl.pallas_call(
        body,
        grid=(1,),
        in_specs=[_full((NC, N_AP, D))],
        out_specs=_full((N_AP, D)),
        out_shape=jax.ShapeDtypeStruct((N_AP, D), f32),
    )(parts)


def _tc_node_update(x, parts, Ws, bs, n, bn):
    g = n // bn

    def body(x_, p, ws, bs_, o):
        pv = p[...]
        o[...] = _relu(_dot(x_[...], ws[...]) + bs_[...] + pv[0] + pv[1])

    return pl.pallas_call(
        body,
        grid=(g,),
        in_specs=[
            _rows((bn, D)),
            pl.BlockSpec((NC, bn, D), lambda i: (0, i, 0)),
            _full((D, D)), _full((1, D)),
        ],
        out_specs=_rows((bn, D)),
        out_shape=jax.ShapeDtypeStruct((n, D), f32),
    )(x, parts, Ws, bs)


def _tc_node_update_head(x, parts, Ws, bs, Wp1, bp1, Wp2, bp2):
    def body(x_, p, ws, bs_, wp1, bp1_, wp2, bp2_, o, opow):
        pv = p[...]
        ue3 = _relu(_dot(x_[...], ws[...]) + bs_[...] + pv[0] + pv[1])
        h = _relu(_dot(ue3, wp1[...]) + bp1_[...])
        opow[...] = jax.nn.sigmoid(_dot(h, wp2[...]) + bp2_[...])
        o[...] = ue3

    return pl.pallas_call(
        body,
        grid=(G_UE,),
        in_specs=[
            _rows((BN_UE, D)),
            pl.BlockSpec((NC, BN_UE, D), lambda i: (0, i, 0)),
            _full((D, D)), _full((1, D)),
            _full((D, 16)), _full((1, 16)), _full((16, 1)), _full((1, 1)),
        ],
        out_specs=[_rows((BN_UE, D)), _rows((BN_UE, 1))],
        out_shape=[
            jax.ShapeDtypeStruct((N_UE, D), f32),
            jax.ShapeDtypeStruct((N_UE, 1), f32),
        ],
    )(x, parts, Ws, bs, Wp1, bp1, Wp2, bp2)


def kernel(x_ue, x_ap, edge_attr_u2a, edge_attr_a2u, src_ue, dst_ap, src_ap, dst_ue,
           We1, be1, Wa1, ba1, Wm1, bm1, Wu1, bu1,
           We2, be2, Wm2, bm2, Ws2, bs2,
           We3, be3, Wm3, bm3, Ws3, bs3,
           Wp1, bp1, Wp2, bp2):
    i_su = src_ue.astype(jnp.int32).reshape(NW, NCH, CH)
    i_da = dst_ap.astype(jnp.int32).reshape(NW, NCH, CH)
    i_sa = src_ap.astype(jnp.int32).reshape(NW, NCH, CH)
    i_du = dst_ue.astype(jnp.int32).reshape(NW, NCH, CH)

    r = lambda b: b.reshape(1, -1)

    xue8, xue32, ue1 = _tc_node1(x_ue, We1[0:1], Wm1[0:1], Wu1[0:1], r(bu1))
    g8 = _sc_gather(xue8, i_su)
    g32 = _sc_gather(xue32, i_su)
    e_u2a, e_a2u, msg1 = _tc_edge1(
        g8, g32, edge_attr_u2a, edge_attr_a2u,
        We1[1:3], r(be1), Wa1, r(ba1), Wm1[1:9], r(bm1))
    ap_parts = _sc_scatter(msg1, i_da, N_AP)
    ap1 = _tc_combine_ap1(ap_parts)

    def het(ue, ap, eu, ea, We, be, Wm, bm):
        gsu = _sc_gather(ue, i_su)
        gda = _sc_gather(ap, i_da)
        gsa = _sc_gather(ap, i_sa)
        gdu = _sc_gather(ue, i_du)
        e1, e2, m_ap, m_ue = _tc_edge_het(
            gsu, gda, gsa, gdu, eu, ea,
            We[0:32], We[32:64], We[64:72], r(be), Wm[0:32], Wm[32:40], r(bm))
        app = _sc_scatter(m_ap, i_da, N_AP)
        uep = _sc_scatter(m_ue, i_du, N_UE)
        return e1, e2, app, uep

    e_u2a, e_a2u, app, uep = het(ue1, ap1, e_u2a, e_a2u, We2, be2, Wm2, bm2)
    ap2 = _tc_node_update(ap1, app, Ws2, r(bs2), N_AP, 1000)
    ue2 = _tc_node_update(ue1, uep, Ws2, r(bs2), N_UE, BN_UE)

    e_u2a, e_a2u, app, uep = het(ue2, ap2, e_u2a, e_a2u, We3, be3, Wm3, bm3)
    ap3 = _tc_node_update(ap2, app, Ws3, r(bs3), N_AP, 1000)
    ue3, power = _tc_node_update_head(ue2, uep, Ws3, r(bs3), Wp1, r(bp1), Wp2, r(bp2))

    ue_out = jnp.concatenate([ue3[:, :1], power], axis=1)
    return (ue_out, ap3, e_u2a, e_a2u)


# speedup vs baseline: 2.6684x; 1.1844x over previous
import functools

import jax
import jax.numpy as jnp
from jax import lax
from jax.experimental import pallas as pl
from jax.experimental.pallas import tpu as pltpu
from jax.experimental.pallas import tpu_sc as plsc

N_UE = 50000
N_AP = 5000
E = 1600000
D = 32
ED = 8

NC = 2
NS = 16
NW = NC * NS
EW = E // NW
CH = 80
NCH = EW // CH

f32 = jnp.float32


def _relu(x):
    return jnp.maximum(x, 0.0)


def _mesh():
    return plsc.VectorSubcoreMesh(
        core_axis_name="c", subcore_axis_name="s", num_cores=NC, num_subcores=NS
    )


NB = 5
NG = NCH // NB


@functools.cache
def _gather_kernel(d):
    def body(table, idx2, out, idx_v, rows_v, gsem, wsem):
        c = lax.axis_index("c")
        s = lax.axis_index("s")
        wid = c * NS + s
        pltpu.sync_copy(idx2.at[wid], idx_v)

        for b in range(NB):
            pltpu.async_copy(table.at[idx_v.at[b]], rows_v.at[b], gsem.at[b])

        def group(gi, carry):
            base = (gi - 1) * NB
            for b in range(NB):
                pltpu.make_async_copy(
                    table.at[idx_v.at[base + b]], rows_v.at[b], gsem.at[b]
                ).wait()
            wdesc = [
                pltpu.async_copy(
                    rows_v.at[b],
                    out.at[pl.ds(wid * EW + (base + b) * CH, CH)],
                    wsem.at[b],
                )
                for b in range(NB)
            ]
            for b in range(NB):
                wdesc[b].wait()

                @pl.when(gi < NG)
                def _fire(b=b):
                    pltpu.async_copy(
                        table.at[idx_v.at[gi * NB + b]], rows_v.at[b], gsem.at[b]
                    )

            return carry

        lax.fori_loop(1, NG + 1, group, 0, unroll=False)

    return pl.kernel(
        body,
        out_type=jax.ShapeDtypeStruct((E, d), f32),
        mesh=_mesh(),
        compiler_params=pltpu.CompilerParams(use_tc_tiling_on_sc=False),
        scratch_types=[
            pltpu.VMEM((NCH, CH), jnp.int32),
            pltpu.VMEM((NB, CH, d), f32),
            pltpu.SemaphoreType.DMA((NB,)),
            pltpu.SemaphoreType.DMA((NB,)),
        ],
    )


def _sc_gather(table, idx2):
    return _gather_kernel(int(table.shape[1]))(table, idx2)


@functools.cache
def _scatter_kernel(n):
    def body(vals, idx2, zeros, out, idx_v, rows_v, acc, isem, rsem, ssem):
        c = lax.axis_index("c")
        s = lax.axis_index("s")
        wid = c * NS + s

        @pl.when(s == 0)
        def _zero():
            pltpu.sync_copy(zeros, acc)

        plsc.subcore_barrier()

        def fire_loads(ci, b):
            pltpu.async_copy(idx2.at[wid, ci], idx_v.at[b], isem.at[b])
            pltpu.async_copy(
                vals.at[pl.ds(wid * EW + ci * CH, CH)], rows_v.at[b], rsem.at[b]
            )

        for b in range(NB):
            fire_loads(b, b)

        def group(gi, carry):
            base = (gi - 1) * NB
            for b in range(NB):
                pltpu.make_async_copy(
                    idx2.at[wid, base + b], idx_v.at[b], isem.at[b]
                ).wait()
                pltpu.make_async_copy(
                    vals.at[pl.ds(wid * EW + (base + b) * CH, CH)],
                    rows_v.at[b], rsem.at[b],
                ).wait()
            sdesc = [
                pltpu.async_copy(
                    rows_v.at[b], acc.at[idx_v.at[b]], ssem.at[b], add=True
                )
                for b in range(NB)
            ]
            for b in range(NB):
                sdesc[b].wait()

                @pl.when(gi < NG)
                def _fire(b=b):
                    fire_loads(gi * NB + b, b)

            return carry

        lax.fori_loop(1, NG + 1, group, 0, unroll=False)
        plsc.subcore_barrier()

        @pl.when(s == 0)
        def _writeout():
            pltpu.sync_copy(acc, out.at[c])

    return pl.kernel(
        body,
        out_type=jax.ShapeDtypeStruct((NC, n, D), f32),
        mesh=_mesh(),
        compiler_params=pltpu.CompilerParams(use_tc_tiling_on_sc=False),
        scratch_types=[
            pltpu.VMEM((NB, CH), jnp.int32),
            pltpu.VMEM((NB, CH, D), f32),
            pltpu.VMEM_SHARED((n, D), f32),
            pltpu.SemaphoreType.DMA((NB,)),
            pltpu.SemaphoreType.DMA((NB,)),
            pltpu.SemaphoreType.DMA((NB,)),
        ],
    )


def _sc_scatter(vals, idx2, n):
    zeros = jnp.zeros((n, D), f32)
    return _scatter_kernel(n)(vals, idx2, zeros)


BE = 3200
GE = E // BE
BN_UE = 2000
G_UE = N_UE // BN_UE


def _full(shape):
    return pl.BlockSpec(shape, lambda i: tuple(0 for _ in shape))


def _rows(shape):
    return pl.BlockSpec(shape, lambda i: (i,) + tuple(0 for _ in shape[1:]))


def _dot(a, b):
    return jax.lax.dot_general(a, b, (((1,), (0,)), ((), ())), preferred_element_type=f32)


def _tc_node1(x_ue, We1r, Wm1r, Wu1, bu1):
    def body(x, we1r, wm1r, wu1, bu1_, o8, o32, oue):
        xv = x[...]
        o8[...] = xv * we1r[...]
        o32[...] = xv * wm1r[...]
        oue[...] = _relu(xv * wu1[...] + bu1_[...])

    return pl.pallas_call(
        body,
        grid=(G_UE,),
        in_specs=[
            _rows((BN_UE, 1)),
            _full((1, ED)), _full((1, D)), _full((1, D)), _full((1, D)),
        ],
        out_specs=[_rows((BN_UE, ED)), _rows((BN_UE, D)), _rows((BN_UE, D))],
        out_shape=[
            jax.ShapeDtypeStruct((N_UE, ED), f32),
            jax.ShapeDtypeStruct((N_UE, D), f32),
            jax.ShapeDtypeStruct((N_UE, D), f32),
        ],
    )(x_ue, We1r, Wm1r, Wu1, bu1)


def _tc_edge1(g8, g32, ea_u2a, ea_a2u, We1b, be1, Wa1, ba1, Wm1b, bm1):
    def body(g8_, g32_, eu, ea, we1b, be1_, wa1, ba1_, wm1b, bm1_, oe1, oe2, om):
        euv = eu[...]
        e1 = _relu(g8_[...] + euv[:, 0:1] * we1b[0:1, :] + euv[:, 1:2] * we1b[1:2, :] + be1_[...])
        eav = ea[...]
        e2 = _relu(eav[:, 0:1] * wa1[0:1, :] + eav[:, 1:2] * wa1[1:2, :] + ba1_[...])
        om[...] = _relu(g32_[...] + _dot(e1, wm1b[...]) + bm1_[...])
        oe1[...] = e1
        oe2[...] = e2

    return pl.pallas_call(
        body,
        grid=(GE,),
        in_specs=[
            _rows((BE, ED)), _rows((BE, D)), _rows((BE, 2)), _rows((BE, 2)),
            _full((2, ED)), _full((1, ED)), _full((2, ED)), _full((1, ED)),
            _full((ED, D)), _full((1, D)),
        ],
        out_specs=[_rows((BE, ED)), _rows((BE, ED)), _rows((BE, D))],
        out_shape=[
            jax.ShapeDtypeStruct((E, ED), f32),
            jax.ShapeDtypeStruct((E, ED), f32),
            jax.ShapeDtypeStruct((E, D), f32),
        ],
    )(g8, g32, ea_u2a, ea_a2u, We1b, be1, Wa1, ba1, Wm1b, bm1)


def _tc_edge_het(gsu, gda, gsa, gdu, eu, ea, WeA, WeB, WeC, be, WmA, WmB, bm):
    def body(gsu_, gda_, gsa_, gdu_, eu_, ea_, weA, weB, weC, be_, wmA, wmB, bm_,
             oe1, oe2, omap, omue):
        e1 = _relu(_dot(gsu_[...], weA[...]) + _dot(gda_[...], weB[...])
                   + _dot(eu_[...], weC[...]) + be_[...])
        e2 = _relu(_dot(gsa_[...], weA[...]) + _dot(gdu_[...], weB[...])
                   + _dot(ea_[...], weC[...]) + be_[...])
        omap[...] = _relu(_dot(gsu_[...], wmA[...]) + _dot(e1, wmB[...]) + bm_[...])
        omue[...] = _relu(_dot(gsa_[...], wmA[...]) + _dot(e2, wmB[...]) + bm_[...])
        oe1[...] = e1
        oe2[...] = e2

    return pl.pallas_call(
        body,
        grid=(GE,),
        in_specs=[
            _rows((BE, D)), _rows((BE, D)), _rows((BE, D)), _rows((BE, D)),
            _rows((BE, ED)), _rows((BE, ED)),
            _full((D, ED)), _full((D, ED)), _full((ED, ED)), _full((1, ED)),
            _full((D, D)), _full((ED, D)), _full((1, D)),
        ],
        out_specs=[_rows((BE, ED)), _rows((BE, ED)), _rows((BE, D)), _rows((BE, D))],
        out_shape=[
            jax.ShapeDtypeStruct((E, ED), f32),
            jax.ShapeDtypeStruct((E, ED), f32),
            jax.ShapeDtypeStruct((E, D), f32),
            jax.ShapeDtypeStruct((E, D), f32),
        ],
    )(gsu, gda, gsa, gdu, eu, ea, WeA, WeB, WeC, be, WmA, WmB, bm)


def _tc_combine_ap1(parts):
    def body(p, o):
        pv = p[...]
        o[...] = pv[0] + pv[1]

    return pl.pallas_call(
        body,
        grid=(1,),
        in_specs=[_full((NC, N_AP, D))],
        out_specs=_full((N_AP, D)),
        out_shape=jax.ShapeDtypeStruct((N_AP, D), f32),
    )(parts)


def _tc_node_update(x, parts, Ws, bs, n, bn):
    g = n // bn

    def body(x_, p, ws, bs_, o):
        pv = p[...]
        o[...] = _relu(_dot(x_[...], ws[...]) + bs_[...] + pv[0] + pv[1])

    return pl.pallas_call(
        body,
        grid=(g,),
        in_specs=[
            _rows((bn, D)),
            pl.BlockSpec((NC, bn, D), lambda i: (0, i, 0)),
            _full((D, D)), _full((1, D)),
        ],
        out_specs=_rows((bn, D)),
        out_shape=jax.ShapeDtypeStruct((n, D), f32),
    )(x, parts, Ws, bs)


def _tc_node_update_head(x, parts, Ws, bs, Wp1, bp1, Wp2, bp2):
    def body(x_, p, ws, bs_, wp1, bp1_, wp2, bp2_, o, opow):
        pv = p[...]
        ue3 = _relu(_dot(x_[...], ws[...]) + bs_[...] + pv[0] + pv[1])
        h = _relu(_dot(ue3, wp1[...]) + bp1_[...])
        opow[...] = jax.nn.sigmoid(_dot(h, wp2[...]) + bp2_[...])
        o[...] = ue3

    return pl.pallas_call(
        body,
        grid=(G_UE,),
        in_specs=[
            _rows((BN_UE, D)),
            pl.BlockSpec((NC, BN_UE, D), lambda i: (0, i, 0)),
            _full((D, D)), _full((1, D)),
            _full((D, 16)), _full((1, 16)), _full((16, 1)), _full((1, 1)),
        ],
        out_specs=[_rows((BN_UE, D)), _rows((BN_UE, 1))],
        out_shape=[
            jax.ShapeDtypeStruct((N_UE, D), f32),
            jax.ShapeDtypeStruct((N_UE, 1), f32),
        ],
    )(x, parts, Ws, bs, Wp1, bp1, Wp2, bp2)


def kernel(x_ue, x_ap, edge_attr_u2a, edge_attr_a2u, src_ue, dst_ap, src_ap, dst_ue,
           We1, be1, Wa1, ba1, Wm1, bm1, Wu1, bu1,
           We2, be2, Wm2, bm2, Ws2, bs2,
           We3, be3, Wm3, bm3, Ws3, bs3,
           Wp1, bp1, Wp2, bp2):
    i_su = src_ue.astype(jnp.int32).reshape(NW, NCH, CH)
    i_da = dst_ap.astype(jnp.int32).reshape(NW, NCH, CH)
    i_sa = src_ap.astype(jnp.int32).reshape(NW, NCH, CH)
    i_du = dst_ue.astype(jnp.int32).reshape(NW, NCH, CH)

    r = lambda b: b.reshape(1, -1)

    xue8, xue32, ue1 = _tc_node1(x_ue, We1[0:1], Wm1[0:1], Wu1[0:1], r(bu1))
    g8 = _sc_gather(xue8, i_su)
    g32 = _sc_gather(xue32, i_su)
    e_u2a, e_a2u, msg1 = _tc_edge1(
        g8, g32, edge_attr_u2a, edge_attr_a2u,
        We1[1:3], r(be1), Wa1, r(ba1), Wm1[1:9], r(bm1))
    ap_parts = _sc_scatter(msg1, i_da, N_AP)
    ap1 = _tc_combine_ap1(ap_parts)

    def het(ue, ap, eu, ea, We, be, Wm, bm):
        gsu = _sc_gather(ue, i_su)
        gda = _sc_gather(ap, i_da)
        gsa = _sc_gather(ap, i_sa)
        gdu = _sc_gather(ue, i_du)
        e1, e2, m_ap, m_ue = _tc_edge_het(
            gsu, gda, gsa, gdu, eu, ea,
            We[0:32], We[32:64], We[64:72], r(be), Wm[0:32], Wm[32:40], r(bm))
        app = _sc_scatter(m_ap, i_da, N_AP)
        uep = _sc_scatter(m_ue, i_du, N_UE)
        return e1, e2, app, uep

    e_u2a, e_a2u, app, uep = het(ue1, ap1, e_u2a, e_a2u, We2, be2, Wm2, bm2)
    ap2 = _tc_node_update(ap1, app, Ws2, r(bs2), N_AP, 1000)
    ue2 = _tc_node_update(ue1, uep, Ws2, r(bs2), N_UE, BN_UE)

    e_u2a, e_a2u, app, uep = het(ue2, ap2, e_u2a, e_a2u, We3, be3, Wm3, bm3)
    ap3 = _tc_node_update(ap2, app, Ws3, r(bs3), N_AP, 1000)
    ue3, power = _tc_node_update_head(ue2, uep, Ws3, r(bs3), Wp1, r(bp1), Wp2, r(bp2))

    ue_out = jnp.concatenate([ue3[:, :1], power], axis=1)
    return (ue_out, ap3, e_u2a, e_a2u)


# speedup vs baseline: 2.6721x; 1.0014x over previous
import functools

import jax
import jax.numpy as jnp
from jax import lax
from jax.experimental import pallas as pl
from jax.experimental.pallas import tpu as pltpu
from jax.experimental.pallas import tpu_sc as plsc

N_UE = 50000
N_AP = 5000
E = 1600000
D = 32
ED = 8

NC = 2
NS = 16
NW = NC * NS
EW = E // NW
CH = 80
NCH = EW // CH

f32 = jnp.float32


def _relu(x):
    return jnp.maximum(x, 0.0)


def _mesh():
    return plsc.VectorSubcoreMesh(
        core_axis_name="c", subcore_axis_name="s", num_cores=NC, num_subcores=NS
    )


NB = 5
NG = NCH // NB
assert NG % 2 == 1


@functools.cache
def _gather_kernel(d):
    def body(table, idx2, out, idx_v, rows_v, gsem, wsem):
        c = lax.axis_index("c")
        s = lax.axis_index("s")
        wid = c * NS + s
        pltpu.sync_copy(idx2.at[wid], idx_v)

        def g_desc(g, k, b):
            return pltpu.make_async_copy(
                table.at[idx_v.at[g * NB + b]], rows_v.at[k, b], gsem.at[k, b])

        def w_desc(g, k, b):
            return pltpu.make_async_copy(
                rows_v.at[k, b],
                out.at[pl.ds(wid * EW + (g * NB + b) * CH, CH)],
                wsem.at[k, b])

        for b in range(NB):
            g_desc(0, 0, b).start()

        def pair(i, carry):
            gA = 2 * i
            gB = gA + 1
            for b in range(NB):
                g_desc(gA, 0, b).wait()

            @pl.when(i > 0)
            def _drain_prev():
                for b in range(NB):
                    w_desc(gA - 1, 1, b).wait()

            for b in range(NB):
                g_desc(gB, 1, b).start()
            for b in range(NB):
                w_desc(gA, 0, b).start()
            for b in range(NB):
                g_desc(gB, 1, b).wait()
            for b in range(NB):
                w_desc(gA, 0, b).wait()
            for b in range(NB):
                g_desc(gA + 2, 0, b).start()
            for b in range(NB):
                w_desc(gB, 1, b).start()
            return carry

        lax.fori_loop(0, (NG - 1) // 2, pair, 0, unroll=False)

        gL = NG - 1
        for b in range(NB):
            g_desc(gL, 0, b).wait()
        for b in range(NB):
            w_desc(gL - 1, 1, b).wait()
        for b in range(NB):
            w_desc(gL, 0, b).start()
        for b in range(NB):
            w_desc(gL, 0, b).wait()

    return pl.kernel(
        body,
        out_type=jax.ShapeDtypeStruct((E, d), f32),
        mesh=_mesh(),
        compiler_params=pltpu.CompilerParams(use_tc_tiling_on_sc=False),
        scratch_types=[
            pltpu.VMEM((NCH, CH), jnp.int32),
            pltpu.VMEM((2, NB, CH, d), f32),
            pltpu.SemaphoreType.DMA((2, NB)),
            pltpu.SemaphoreType.DMA((2, NB)),
        ],
    )


def _sc_gather(table, idx2):
    return _gather_kernel(int(table.shape[1]))(table, idx2)


@functools.cache
def _scatter_kernel(n):
    def body(vals, idx2, zeros, out, idx_v, rows_v, acc, isem, rsem, ssem):
        c = lax.axis_index("c")
        s = lax.axis_index("s")
        wid = c * NS + s

        def i_desc(g, k, b):
            return pltpu.make_async_copy(
                idx2.at[wid, g * NB + b], idx_v.at[k, b], isem.at[k, b])

        def r_desc(g, k, b):
            return pltpu.make_async_copy(
                vals.at[pl.ds(wid * EW + (g * NB + b) * CH, CH)],
                rows_v.at[k, b], rsem.at[k, b])

        def s_desc(k, b):
            return pltpu.make_async_copy(
                rows_v.at[k, b], acc.at[idx_v.at[k, b]], ssem.at[k, b])

        for b in range(NB):
            i_desc(0, 0, b).start()
            r_desc(0, 0, b).start()

        @pl.when(s == 0)
        def _zero():
            pltpu.sync_copy(zeros, acc)

        plsc.subcore_barrier()

        def pair(i, carry):
            gA = 2 * i
            gB = gA + 1
            for b in range(NB):
                i_desc(gA, 0, b).wait()
                r_desc(gA, 0, b).wait()

            @pl.when(i > 0)
            def _drain_prev():
                for b in range(NB):
                    s_desc(1, b).wait()

            for b in range(NB):
                i_desc(gB, 1, b).start()
                r_desc(gB, 1, b).start()
            for b in range(NB):
                s_desc(0, b).start(add=True)
            for b in range(NB):
                i_desc(gB, 1, b).wait()
                r_desc(gB, 1, b).wait()
            for b in range(NB):
                s_desc(0, b).wait()
            for b in range(NB):
                i_desc(gA + 2, 0, b).start()
                r_desc(gA + 2, 0, b).start()
            for b in range(NB):
                s_desc(1, b).start(add=True)
            return carry

        lax.fori_loop(0, (NG - 1) // 2, pair, 0, unroll=False)

        gL = NG - 1
        for b in range(NB):
            i_desc(gL, 0, b).wait()
            r_desc(gL, 0, b).wait()
        for b in range(NB):
            s_desc(1, b).wait()
        for b in range(NB):
            s_desc(0, b).start(add=True)
        for b in range(NB):
            s_desc(0, b).wait()
        plsc.subcore_barrier()

        @pl.when(s == 0)
        def _writeout():
            pltpu.sync_copy(acc, out.at[c])

    return pl.kernel(
        body,
        out_type=jax.ShapeDtypeStruct((NC, n, D), f32),
        mesh=_mesh(),
        compiler_params=pltpu.CompilerParams(use_tc_tiling_on_sc=False),
        scratch_types=[
            pltpu.VMEM((2, NB, CH), jnp.int32),
            pltpu.VMEM((2, NB, CH, D), f32),
            pltpu.VMEM_SHARED((n, D), f32),
            pltpu.SemaphoreType.DMA((2, NB)),
            pltpu.SemaphoreType.DMA((2, NB)),
            pltpu.SemaphoreType.DMA((2, NB)),
        ],
    )


def _sc_scatter(vals, idx2, n):
    zeros = jnp.zeros((n, D), f32)
    return _scatter_kernel(n)(vals, idx2, zeros)


BE = 3200
GE = E // BE
BN_UE = 2000
G_UE = N_UE // BN_UE


def _full(shape):
    return pl.BlockSpec(shape, lambda i: tuple(0 for _ in shape))


def _rows(shape):
    return pl.BlockSpec(shape, lambda i: (i,) + tuple(0 for _ in shape[1:]))


def _dot(a, b):
    return jax.lax.dot_general(a, b, (((1,), (0,)), ((), ())), preferred_element_type=f32)


def _tc_node1(x_ue, We1r, Wm1r, Wu1, bu1):
    def body(x, we1r, wm1r, wu1, bu1_, o8, o32, oue):
        xv = x[...]
        o8[...] = xv * we1r[...]
        o32[...] = xv * wm1r[...]
        oue[...] = _relu(xv * wu1[...] + bu1_[...])

    return pl.pallas_call(
        body,
        grid=(G_UE,),
        in_specs=[
            _rows((BN_UE, 1)),
            _full((1, ED)), _full((1, D)), _full((1, D)), _full((1, D)),
        ],
        out_specs=[_rows((BN_UE, ED)), _rows((BN_UE, D)), _rows((BN_UE, D))],
        out_shape=[
            jax.ShapeDtypeStruct((N_UE, ED), f32),
            jax.ShapeDtypeStruct((N_UE, D), f32),
            jax.ShapeDtypeStruct((N_UE, D), f32),
        ],
    )(x_ue, We1r, Wm1r, Wu1, bu1)


def _tc_edge1(g8, g32, ea_u2a, ea_a2u, We1b, be1, Wa1, ba1, Wm1b, bm1):
    def body(g8_, g32_, eu, ea, we1b, be1_, wa1, ba1_, wm1b, bm1_, oe1, oe2, om):
        euv = eu[...]
        e1 = _relu(g8_[...] + euv[:, 0:1] * we1b[0:1, :] + euv[:, 1:2] * we1b[1:2, :] + be1_[...])
        eav = ea[...]
        e2 = _relu(eav[:, 0:1] * wa1[0:1, :] + eav[:, 1:2] * wa1[1:2, :] + ba1_[...])
        om[...] = _relu(g32_[...] + _dot(e1, wm1b[...]) + bm1_[...])
        oe1[...] = e1
        oe2[...] = e2

    return pl.pallas_call(
        body,
        grid=(GE,),
        in_specs=[
            _rows((BE, ED)), _rows((BE, D)), _rows((BE, 2)), _rows((BE, 2)),
            _full((2, ED)), _full((1, ED)), _full((2, ED)), _full((1, ED)),
            _full((ED, D)), _full((1, D)),
        ],
        out_specs=[_rows((BE, ED)), _rows((BE, ED)), _rows((BE, D))],
        out_shape=[
            jax.ShapeDtypeStruct((E, ED), f32),
            jax.ShapeDtypeStruct((E, ED), f32),
            jax.ShapeDtypeStruct((E, D), f32),
        ],
    )(g8, g32, ea_u2a, ea_a2u, We1b, be1, Wa1, ba1, Wm1b, bm1)


def _tc_edge_het(gsu, gda, gsa, gdu, eu, ea, WeA, WeB, WeC, be, WmA, WmB, bm):
    def body(gsu_, gda_, gsa_, gdu_, eu_, ea_, weA, weB, weC, be_, wmA, wmB, bm_,
             oe1, oe2, omap, omue):
        e1 = _relu(_dot(gsu_[...], weA[...]) + _dot(gda_[...], weB[...])
                   + _dot(eu_[...], weC[...]) + be_[...])
        e2 = _relu(_dot(gsa_[...], weA[...]) + _dot(gdu_[...], weB[...])
                   + _dot(ea_[...], weC[...]) + be_[...])
        omap[...] = _relu(_dot(gsu_[...], wmA[...]) + _dot(e1, wmB[...]) + bm_[...])
        omue[...] = _relu(_dot(gsa_[...], wmA[...]) + _dot(e2, wmB[...]) + bm_[...])
        oe1[...] = e1
        oe2[...] = e2

    return pl.pallas_call(
        body,
        grid=(GE,),
        in_specs=[
            _rows((BE, D)), _rows((BE, D)), _rows((BE, D)), _rows((BE, D)),
            _rows((BE, ED)), _rows((BE, ED)),
            _full((D, ED)), _full((D, ED)), _full((ED, ED)), _full((1, ED)),
            _full((D, D)), _full((ED, D)), _full((1, D)),
        ],
        out_specs=[_rows((BE, ED)), _rows((BE, ED)), _rows((BE, D)), _rows((BE, D))],
        out_shape=[
            jax.ShapeDtypeStruct((E, ED), f32),
            jax.ShapeDtypeStruct((E, ED), f32),
            jax.ShapeDtypeStruct((E, D), f32),
            jax.ShapeDtypeStruct((E, D), f32),
        ],
    )(gsu, gda, gsa, gdu, eu, ea, WeA, WeB, WeC, be, WmA, WmB, bm)


def _tc_combine_ap1(parts):
    def body(p, o):
        pv = p[...]
        o[...] = pv[0] + pv[1]

    return pl.pallas_call(
        body,
        grid=(1,),
        in_specs=[_full((NC, N_AP, D))],
        out_specs=_full((N_AP, D)),
        out_shape=jax.ShapeDtypeStruct((N_AP, D), f32),
    )(parts)


def _tc_node_update(x, parts, Ws, bs, n, bn):
    g = n // bn

    def body(x_, p, ws, bs_, o):
        pv = p[...]
        o[...] = _relu(_dot(x_[...], ws[...]) + bs_[...] + pv[0] + pv[1])

    return pl.pallas_call(
        body,
        grid=(g,),
        in_specs=[
            _rows((bn, D)),
            pl.BlockSpec((NC, bn, D), lambda i: (0, i, 0)),
            _full((D, D)), _full((1, D)),
        ],
        out_specs=_rows((bn, D)),
        out_shape=jax.ShapeDtypeStruct((n, D), f32),
    )(x, parts, Ws, bs)


def _tc_node_update_head(x, parts, Ws, bs, Wp1, bp1, Wp2, bp2):
    def body(x_, p, ws, bs_, wp1, bp1_, wp2, bp2_, o, opow):
        pv = p[...]
        ue3 = _relu(_dot(x_[...], ws[...]) + bs_[...] + pv[0] + pv[1])
        h = _relu(_dot(ue3, wp1[...]) + bp1_[...])
        opow[...] = jax.nn.sigmoid(_dot(h, wp2[...]) + bp2_[...])
        o[...] = ue3

    return pl.pallas_call(
        body,
        grid=(G_UE,),
        in_specs=[
            _rows((BN_UE, D)),
            pl.BlockSpec((NC, BN_UE, D), lambda i: (0, i, 0)),
            _full((D, D)), _full((1, D)),
            _full((D, 16)), _full((1, 16)), _full((16, 1)), _full((1, 1)),
        ],
        out_specs=[_rows((BN_UE, D)), _rows((BN_UE, 1))],
        out_shape=[
            jax.ShapeDtypeStruct((N_UE, D), f32),
            jax.ShapeDtypeStruct((N_UE, 1), f32),
        ],
    )(x, parts, Ws, bs, Wp1, bp1, Wp2, bp2)


def kernel(x_ue, x_ap, edge_attr_u2a, edge_attr_a2u, src_ue, dst_ap, src_ap, dst_ue,
           We1, be1, Wa1, ba1, Wm1, bm1, Wu1, bu1,
           We2, be2, Wm2, bm2, Ws2, bs2,
           We3, be3, Wm3, bm3, Ws3, bs3,
           Wp1, bp1, Wp2, bp2):
    i_su = src_ue.astype(jnp.int32).reshape(NW, NCH, CH)
    i_da = dst_ap.astype(jnp.int32).reshape(NW, NCH, CH)
    i_sa = src_ap.astype(jnp.int32).reshape(NW, NCH, CH)
    i_du = dst_ue.astype(jnp.int32).reshape(NW, NCH, CH)

    r = lambda b: b.reshape(1, -1)

    xue8, xue32, ue1 = _tc_node1(x_ue, We1[0:1], Wm1[0:1], Wu1[0:1], r(bu1))
    g8 = _sc_gather(xue8, i_su)
    g32 = _sc_gather(xue32, i_su)
    e_u2a, e_a2u, msg1 = _tc_edge1(
        g8, g32, edge_attr_u2a, edge_attr_a2u,
        We1[1:3], r(be1), Wa1, r(ba1), Wm1[1:9], r(bm1))
    ap_parts = _sc_scatter(msg1, i_da, N_AP)
    ap1 = _tc_combine_ap1(ap_parts)

    def het(ue, ap, eu, ea, We, be, Wm, bm):
        gsu = _sc_gather(ue, i_su)
        gda = _sc_gather(ap, i_da)
        gsa = _sc_gather(ap, i_sa)
        gdu = _sc_gather(ue, i_du)
        e1, e2, m_ap, m_ue = _tc_edge_het(
            gsu, gda, gsa, gdu, eu, ea,
            We[0:32], We[32:64], We[64:72], r(be), Wm[0:32], Wm[32:40], r(bm))
        app = _sc_scatter(m_ap, i_da, N_AP)
        uep = _sc_scatter(m_ue, i_du, N_UE)
        return e1, e2, app, uep

    e_u2a, e_a2u, app, uep = het(ue1, ap1, e_u2a, e_a2u, We2, be2, Wm2, bm2)
    ap2 = _tc_node_update(ap1, app, Ws2, r(bs2), N_AP, 1000)
    ue2 = _tc_node_update(ue1, uep, Ws2, r(bs2), N_UE, BN_UE)

    e_u2a, e_a2u, app, uep = het(ue2, ap2, e_u2a, e_a2u, We3, be3, Wm3, bm3)
    ap3 = _tc_node_update(ap2, app, Ws3, r(bs3), N_AP, 1000)
    ue3, power = _tc_node_update_head(ue2, uep, Ws3, r(bs3), Wp1, r(bp1), Wp2, r(bp2))

    ue_out = jnp.concatenate([ue3[:, :1], power], axis=1)
    return (ue_out, ap3, e_u2a, e_a2u)


# speedup vs baseline: 2.7574x; 1.0319x over previous
import functools

import jax
import jax.numpy as jnp
from jax import lax
from jax.experimental import pallas as pl
from jax.experimental.pallas import tpu as pltpu
from jax.experimental.pallas import tpu_sc as plsc

N_UE = 50000
N_AP = 5000
E = 1600000
D = 32
ED = 8

NC = 2
NS = 16
NW = NC * NS
EW = E // NW
CH = 80
NCH = EW // CH
NB = 5
NG = NCH // NB
assert NG % 2 == 1

f32 = jnp.float32


def _relu(x):
    return jnp.maximum(x, 0.0)


def _b16(x):
    return x.astype(jnp.bfloat16).astype(jnp.float32)


def _mesh():
    return plsc.VectorSubcoreMesh(
        core_axis_name="c", subcore_axis_name="s", num_cores=NC, num_subcores=NS
    )


@functools.cache
def _gather_kernel(d):
    def body(table, idx2, out, idx_v, rows_v, gsem, wsem):
        c = lax.axis_index("c")
        s = lax.axis_index("s")
        wid = c * NS + s
        pltpu.sync_copy(idx2.at[wid], idx_v)

        def g_desc(g, k, b):
            return pltpu.make_async_copy(
                table.at[idx_v.at[g * NB + b]], rows_v.at[k, b], gsem.at[k, b])

        def w_desc(g, k, b):
            return pltpu.make_async_copy(
                rows_v.at[k, b],
                out.at[pl.ds(wid * EW + (g * NB + b) * CH, CH)],
                wsem.at[k, b])

        for b in range(NB):
            g_desc(0, 0, b).start()

        def pair(i, carry):
            gA = 2 * i
            gB = gA + 1
            for b in range(NB):
                g_desc(gA, 0, b).wait()

            @pl.when(i > 0)
            def _drain_prev():
                for b in range(NB):
                    w_desc(gA - 1, 1, b).wait()

            for b in range(NB):
                g_desc(gB, 1, b).start()
            for b in range(NB):
                w_desc(gA, 0, b).start()
            for b in range(NB):
                g_desc(gB, 1, b).wait()
            for b in range(NB):
                w_desc(gA, 0, b).wait()
            for b in range(NB):
                g_desc(gA + 2, 0, b).start()
            for b in range(NB):
                w_desc(gB, 1, b).start()
            return carry

        lax.fori_loop(0, (NG - 1) // 2, pair, 0, unroll=False)

        gL = NG - 1
        for b in range(NB):
            g_desc(gL, 0, b).wait()
        for b in range(NB):
            w_desc(gL - 1, 1, b).wait()
        for b in range(NB):
            w_desc(gL, 0, b).start()
        for b in range(NB):
            w_desc(gL, 0, b).wait()

    return pl.kernel(
        body,
        out_type=jax.ShapeDtypeStruct((E, d), f32),
        mesh=_mesh(),
        compiler_params=pltpu.CompilerParams(use_tc_tiling_on_sc=False),
        scratch_types=[
            pltpu.VMEM((NCH, CH), jnp.int32),
            pltpu.VMEM((2, NB, CH, d), f32),
            pltpu.SemaphoreType.DMA((2, NB)),
            pltpu.SemaphoreType.DMA((2, NB)),
        ],
    )


def _sc_gather(table, idx2):
    return _gather_kernel(int(table.shape[1]))(table, idx2)


@functools.cache
def _m_scatter_kernel(n):
    def body(tableM, gidx3, pre, sidx3, zeros, out,
             gi_v, si_v, grows, prows, acc, gisem, sisem, gsem, psem, ssem):
        c = lax.axis_index("c")
        s = lax.axis_index("s")
        wid = c * NS + s

        def gi_d(g, b):
            return pltpu.make_async_copy(
                gidx3.at[wid, g * NB + b], gi_v.at[b], gisem.at[b])

        def si_d(g, b):
            return pltpu.make_async_copy(
                sidx3.at[wid, g * NB + b], si_v.at[b], sisem.at[b])

        def p_d(g, b):
            return pltpu.make_async_copy(
                pre.at[pl.ds(wid * EW + (g * NB + b) * CH, CH)],
                prows.at[b], psem.at[b])

        def g_d(b):
            return pltpu.make_async_copy(
                tableM.at[gi_v.at[b]], grows.at[b], gsem.at[b])

        def s_d(b):
            return pltpu.make_async_copy(
                prows.at[b], acc.at[si_v.at[b]], ssem.at[b])

        for b in range(NB):
            gi_d(0, b).start()
            si_d(0, b).start()
            p_d(0, b).start()

        @pl.when(s == 0)
        def _zero():
            pltpu.sync_copy(zeros, acc)

        plsc.subcore_barrier()
        ngrp = NCH // NB

        def group(g, carry):
            for b in range(NB):
                gi_d(g, b).wait()
            for b in range(NB):
                g_d(b).start()
            for b in range(NB):
                g_d(b).wait()
                p_d(g, b).wait()
                si_d(g, b).wait()
            for b in range(NB):
                def row(r, cc, b=b):
                    for h in range(2):
                        v = (grows[b, r, pl.ds(h * 16, 16)]
                             + prows[b, r, pl.ds(h * 16, 16)])
                        prows[b, r, pl.ds(h * 16, 16)] = jnp.maximum(v, 0.0)
                    return cc

                lax.fori_loop(0, CH, row, 0, unroll=False)
            for b in range(NB):
                s_d(b).start(add=True)
            for b in range(NB):
                s_d(b).wait()

            @pl.when(g + 1 < ngrp)
            def _next():
                for b in range(NB):
                    gi_d(g + 1, b).start()
                    si_d(g + 1, b).start()
                    p_d(g + 1, b).start()

            return carry

        lax.fori_loop(0, ngrp, group, 0, unroll=False)
        plsc.subcore_barrier()

        @pl.when(s == 0)
        def _writeout():
            pltpu.sync_copy(acc, out.at[c])

    return pl.kernel(
        body,
        out_type=jax.ShapeDtypeStruct((NC, n, D), f32),
        mesh=_mesh(),
        compiler_params=pltpu.CompilerParams(use_tc_tiling_on_sc=False),
        scratch_types=[
            pltpu.VMEM((NB, CH), jnp.int32),
            pltpu.VMEM((NB, CH), jnp.int32),
            pltpu.VMEM((NB, CH, D), f32),
            pltpu.VMEM((NB, CH, D), f32),
            pltpu.VMEM_SHARED((n, D), f32),
            pltpu.SemaphoreType.DMA((NB,)),
            pltpu.SemaphoreType.DMA((NB,)),
            pltpu.SemaphoreType.DMA((NB,)),
            pltpu.SemaphoreType.DMA((NB,)),
            pltpu.SemaphoreType.DMA((NB,)),
        ],
    )


def _sc_m_scatter(tableM, gidx3, pre, sidx3, n):
    zeros = jnp.zeros((n, D), f32)
    return _m_scatter_kernel(n)(tableM, gidx3, pre, sidx3, zeros)


BE = 3200
GE = E // BE
BN_UE = 2000
G_UE = N_UE // BN_UE
BN_AP = 1000
G_AP = N_AP // BN_AP


def _full(shape):
    return pl.BlockSpec(shape, lambda i: tuple(0 for _ in shape))


def _rows(shape):
    return pl.BlockSpec(shape, lambda i: (i,) + tuple(0 for _ in shape[1:]))


def _dot(a, b):
    return jax.lax.dot_general(a, b, (((1,), (0,)), ((), ())), preferred_element_type=f32)


def _sds(n, d):
    return jax.ShapeDtypeStruct((n, d), f32)


def _tc_node1(x_ue, We1r, Wm1r, Wu1, bu1, WA, WB, WM):
    def body(x, we1r, wm1r, wu1, bu1_, wa, wb, wm, o8, o32, oue, oa, ob, om):
        xv = _b16(x[...])
        o8[...] = xv * _b16(we1r[...])
        o32[...] = xv * _b16(wm1r[...])
        ue1 = _relu(xv * _b16(wu1[...]) + bu1_[...])
        oue[...] = ue1
        oa[...] = _dot(ue1, wa[...])
        ob[...] = _dot(ue1, wb[...])
        om[...] = _dot(ue1, wm[...])

    return pl.pallas_call(
        body,
        grid=(G_UE,),
        in_specs=[
            _rows((BN_UE, 1)),
            _full((1, ED)), _full((1, D)), _full((1, D)), _full((1, D)),
            _full((D, ED)), _full((D, ED)), _full((D, D)),
        ],
        out_specs=[_rows((BN_UE, ED)), _rows((BN_UE, D)), _rows((BN_UE, D)),
                   _rows((BN_UE, ED)), _rows((BN_UE, ED)), _rows((BN_UE, D))],
        out_shape=[_sds(N_UE, ED), _sds(N_UE, D), _sds(N_UE, D),
                   _sds(N_UE, ED), _sds(N_UE, ED), _sds(N_UE, D)],
    )(x_ue, We1r, Wm1r, Wu1, bu1, WA, WB, WM)


def _tc_edge1(g8, ea_u2a, ea_a2u, We1b, be1, Wa1, ba1, Wm1b, bm1):
    def body(g8_, eu, ea, we1b, be1_, wa1, ba1_, wm1b, bm1_, oe1, oe2, opm):
        euv = _b16(eu[...])
        w1 = _b16(we1b[...])
        e1 = _relu(g8_[...] + euv[:, 0:1] * w1[0:1, :]
                   + euv[:, 1:2] * w1[1:2, :] + be1_[...])
        eav = _b16(ea[...])
        wa = _b16(wa1[...])
        oe2[...] = _relu(eav[:, 0:1] * wa[0:1, :] + eav[:, 1:2] * wa[1:2, :]
                         + ba1_[...])
        opm[...] = _dot(e1, wm1b[...]) + bm1_[...]
        oe1[...] = e1

    return pl.pallas_call(
        body,
        grid=(GE,),
        in_specs=[
            _rows((BE, ED)), _rows((BE, 2)), _rows((BE, 2)),
            _full((2, ED)), _full((1, ED)), _full((2, ED)), _full((1, ED)),
            _full((ED, D)), _full((1, D)),
        ],
        out_specs=[_rows((BE, ED)), _rows((BE, ED)), _rows((BE, D))],
        out_shape=[_sds(E, ED), _sds(E, ED), _sds(E, D)],
    )(g8, ea_u2a, ea_a2u, We1b, be1, Wa1, ba1, Wm1b, bm1)


def _tc_ap1(parts, WA, WB, WM):
    def body(p, wa, wb, wm, oap, oa, ob, om):
        pv = p[...]
        ap1 = pv[0] + pv[1]
        oap[...] = ap1
        oa[...] = _dot(ap1, wa[...])
        ob[...] = _dot(ap1, wb[...])
        om[...] = _dot(ap1, wm[...])

    return pl.pallas_call(
        body,
        grid=(G_AP,),
        in_specs=[
            pl.BlockSpec((NC, BN_AP, D), lambda i: (0, i, 0)),
            _full((D, ED)), _full((D, ED)), _full((D, D)),
        ],
        out_specs=[_rows((BN_AP, D)), _rows((BN_AP, ED)), _rows((BN_AP, ED)),
                   _rows((BN_AP, D))],
        out_shape=[_sds(N_AP, D), _sds(N_AP, ED), _sds(N_AP, ED), _sds(N_AP, D)],
    )(parts, WA, WB, WM)


def _tc_edge_het(gA1, gB1, gA2, gB2, eu, ea, WeC, be, WmB, bm):
    def body(a1, b1, a2, b2, eu_, ea_, weC, be_, wmB, bm_, oe1, oe2, op1, op2):
        e1 = _relu(a1[...] + b1[...] + _dot(eu_[...], weC[...]) + be_[...])
        e2 = _relu(a2[...] + b2[...] + _dot(ea_[...], weC[...]) + be_[...])
        op1[...] = _dot(e1, wmB[...]) + bm_[...]
        op2[...] = _dot(e2, wmB[...]) + bm_[...]
        oe1[...] = e1
        oe2[...] = e2

    return pl.pallas_call(
        body,
        grid=(GE,),
        in_specs=[
            _rows((BE, ED)), _rows((BE, ED)), _rows((BE, ED)), _rows((BE, ED)),
            _rows((BE, ED)), _rows((BE, ED)),
            _full((ED, ED)), _full((1, ED)), _full((ED, D)), _full((1, D)),
        ],
        out_specs=[_rows((BE, ED)), _rows((BE, ED)), _rows((BE, D)), _rows((BE, D))],
        out_shape=[_sds(E, ED), _sds(E, ED), _sds(E, D), _sds(E, D)],
    )(gA1, gB1, gA2, gB2, eu, ea, WeC, be, WmB, bm)


def _tc_node_up_pre(x, parts, Ws, bs, WA, WB, WM, n, bn):
    g = n // bn

    def body(x_, p, ws, bs_, wa, wb, wm, ox, oa, ob, om):
        pv = p[...]
        xn = _relu(_dot(x_[...], ws[...]) + bs_[...] + pv[0] + pv[1])
        ox[...] = xn
        oa[...] = _dot(xn, wa[...])
        ob[...] = _dot(xn, wb[...])
        om[...] = _dot(xn, wm[...])

    return pl.pallas_call(
        body,
        grid=(g,),
        in_specs=[
            _rows((bn, D)),
            pl.BlockSpec((NC, bn, D), lambda i: (0, i, 0)),
            _full((D, D)), _full((1, D)),
            _full((D, ED)), _full((D, ED)), _full((D, D)),
        ],
        out_specs=[_rows((bn, D)), _rows((bn, ED)), _rows((bn, ED)), _rows((bn, D))],
        out_shape=[_sds(n, D), _sds(n, ED), _sds(n, ED), _sds(n, D)],
    )(x, parts, Ws, bs, WA, WB, WM)


def _tc_node_up(x, parts, Ws, bs, n, bn):
    g = n // bn

    def body(x_, p, ws, bs_, o):
        pv = p[...]
        o[...] = _relu(_dot(x_[...], ws[...]) + bs_[...] + pv[0] + pv[1])

    return pl.pallas_call(
        body,
        grid=(g,),
        in_specs=[
            _rows((bn, D)),
            pl.BlockSpec((NC, bn, D), lambda i: (0, i, 0)),
            _full((D, D)), _full((1, D)),
        ],
        out_specs=_rows((bn, D)),
        out_shape=_sds(n, D),
    )(x, parts, Ws, bs)


def _tc_node_up_head(x, parts, Ws, bs, Wp1, bp1, Wp2, bp2):
    def body(x_, p, ws, bs_, wp1, bp1_, wp2, bp2_, o, opow):
        pv = p[...]
        ue3 = _relu(_dot(x_[...], ws[...]) + bs_[...] + pv[0] + pv[1])
        h = _relu(_dot(ue3, wp1[...]) + bp1_[...])
        opow[...] = jax.nn.sigmoid(_dot(h, wp2[...]) + bp2_[...])
        o[...] = ue3

    return pl.pallas_call(
        body,
        grid=(G_UE,),
        in_specs=[
            _rows((BN_UE, D)),
            pl.BlockSpec((NC, BN_UE, D), lambda i: (0, i, 0)),
            _full((D, D)), _full((1, D)),
            _full((D, 16)), _full((1, 16)), _full((16, 1)), _full((1, 1)),
        ],
        out_specs=[_rows((BN_UE, D)), _rows((BN_UE, 1))],
        out_shape=[_sds(N_UE, D), _sds(N_UE, 1)],
    )(x, parts, Ws, bs, Wp1, bp1, Wp2, bp2)


def kernel(x_ue, x_ap, edge_attr_u2a, edge_attr_a2u, src_ue, dst_ap, src_ap, dst_ue,
           We1, be1, Wa1, ba1, Wm1, bm1, Wu1, bu1,
           We2, be2, Wm2, bm2, Ws2, bs2,
           We3, be3, Wm3, bm3, Ws3, bs3,
           Wp1, bp1, Wp2, bp2):
    i_su = src_ue.astype(jnp.int32).reshape(NW, NCH, CH)
    i_da = dst_ap.astype(jnp.int32).reshape(NW, NCH, CH)
    i_sa = src_ap.astype(jnp.int32).reshape(NW, NCH, CH)
    i_du = dst_ue.astype(jnp.int32).reshape(NW, NCH, CH)

    r = lambda b: b.reshape(1, -1)

    xue8, xue32, ue1, ueA2, ueB2, ueM2 = _tc_node1(
        x_ue, We1[0:1], Wm1[0:1], Wu1[0:1], r(bu1),
        We2[0:32], We2[32:64], Wm2[0:32])
    g8 = _sc_gather(xue8, i_su)
    e_u2a, e_a2u, preM1 = _tc_edge1(
        g8, edge_attr_u2a, edge_attr_a2u,
        We1[1:3], r(be1), Wa1, r(ba1), Wm1[1:9], r(bm1))
    apP = _sc_m_scatter(xue32, i_su, preM1, i_da, N_AP)
    ap1, apA2, apB2, apM2 = _tc_ap1(apP, We2[0:32], We2[32:64], Wm2[0:32])

    gA1 = _sc_gather(ueA2, i_su)
    gB1 = _sc_gather(apB2, i_da)
    gA2 = _sc_gather(apA2, i_sa)
    gB2 = _sc_gather(ueB2, i_du)
    e_u2a, e_a2u, preMap, preMue = _tc_edge_het(
        gA1, gB1, gA2, gB2, e_u2a, e_a2u,
        We2[64:72], r(be2), Wm2[32:40], r(bm2))
    apP = _sc_m_scatter(ueM2, i_su, preMap, i_da, N_AP)
    ueP = _sc_m_scatter(apM2, i_sa, preMue, i_du, N_UE)
    ap2, apA3, apB3, apM3 = _tc_node_up_pre(
        ap1, apP, Ws2, r(bs2), We3[0:32], We3[32:64], Wm3[0:32], N_AP, BN_AP)
    ue2, ueA3, ueB3, ueM3 = _tc_node_up_pre(
        ue1, ueP, Ws2, r(bs2), We3[0:32], We3[32:64], Wm3[0:32], N_UE, BN_UE)

    gA1 = _sc_gather(ueA3, i_su)
    gB1 = _sc_gather(apB3, i_da)
    gA2 = _sc_gather(apA3, i_sa)
    gB2 = _sc_gather(ueB3, i_du)
    e_u2a, e_a2u, preMap, preMue = _tc_edge_het(
        gA1, gB1, gA2, gB2, e_u2a, e_a2u,
        We3[64:72], r(be3), Wm3[32:40], r(bm3))
    apP = _sc_m_scatter(ueM3, i_su, preMap, i_da, N_AP)
    ueP = _sc_m_scatter(apM3, i_sa, preMue, i_du, N_UE)
    ap3 = _tc_node_up(ap2, apP, Ws3, r(bs3), N_AP, BN_AP)
    ue3, power = _tc_node_up_head(ue2, ueP, Ws3, r(bs3), Wp1, r(bp1), Wp2, r(bp2))

    ue_out = jnp.concatenate([ue3[:, :1], power], axis=1)
    return (ue_out, ap3, e_u2a, e_a2u)
